# baseline (speedup 1.0000x reference)
import jax
import jax.numpy as jnp
from jax.experimental import pallas as pl
from jax.experimental.pallas import tpu as pltpu

_BATCH = 4096
_DIM = 128
_NDMA = 4
_ROWS = _BATCH // _NDMA


def _copy_body(x_ref, o_ref, sems):
    for i in range(_NDMA):
        pltpu.make_async_copy(
            x_ref.at[pl.ds(i * _ROWS, _ROWS)],
            o_ref.at[pl.ds(i * _ROWS, _ROWS)],
            sems.at[i],
        ).start()
    for i in range(_NDMA):
        pltpu.make_async_copy(
            x_ref.at[pl.ds(i * _ROWS, _ROWS)],
            o_ref.at[pl.ds(i * _ROWS, _ROWS)],
            sems.at[i],
        ).wait()


def kernel(initial_space, finite_space, time_embedding):
    return pl.pallas_call(
        _copy_body,
        in_specs=[pl.BlockSpec(memory_space=pl.ANY)],
        out_specs=pl.BlockSpec(memory_space=pl.ANY),
        out_shape=jax.ShapeDtypeStruct((_BATCH, _DIM), jnp.float32),
        scratch_shapes=[pltpu.SemaphoreType.DMA((_NDMA,))],
    )(initial_space)


# speedup vs baseline: 23.2627x; 23.2627x over previous
import jax
import jax.numpy as jnp
from jax.experimental import pallas as pl
from jax.experimental.pallas import tpu as pltpu

_BATCH = 4096
_DIM = 128
_NBLK = 2
_ROWS = _BATCH // _NBLK


def _copy_body(x_ref, o_ref):
    o_ref[...] = x_ref[...]


def kernel(initial_space, finite_space, time_embedding):
    return pl.pallas_call(
        _copy_body,
        grid=(_NBLK,),
        in_specs=[pl.BlockSpec((_ROWS, _DIM), lambda i: (i, 0))],
        out_specs=pl.BlockSpec((_ROWS, _DIM), lambda i: (i, 0)),
        out_shape=jax.ShapeDtypeStruct((_BATCH, _DIM), jnp.float32),
        compiler_params=pltpu.CompilerParams(
            dimension_semantics=("arbitrary",),
        ),
    )(initial_space)
